# baseline (speedup 1.0000x reference)
import functools

import jax
import jax.numpy as jnp
from jax import lax
from jax.experimental import pallas as pl
from jax.experimental.pallas import tpu as pltpu
from jax.experimental.pallas import tpu_sc as plsc

LANES = 16


@functools.lru_cache(maxsize=None)
def _build(batch: int, seq_len: int, vocab: int, embed: int):
    info = plsc.get_sparse_core_info()
    nw = info.num_cores * info.num_subcores
    assert batch % nw == 0
    bpw = batch // nw
    vecs = embed // LANES

    mesh = plsc.VectorSubcoreMesh(core_axis_name="c", subcore_axis_name="s")

    @functools.partial(
        pl.kernel,
        out_type=jax.ShapeDtypeStruct((seq_len, embed, batch), jnp.float32),
        mesh=mesh,
        scratch_types=[
            pltpu.VMEM((bpw,), jnp.int32),
            pltpu.VMEM((bpw,), jnp.int32),
            pltpu.VMEM((bpw, embed), jnp.float32),
            pltpu.VMEM((bpw, embed), jnp.float32),
            pltpu.VMEM((embed, bpw), jnp.float32),
            pltpu.VMEM((embed, bpw), jnp.float32),
            pltpu.VMEM((seq_len, embed), jnp.float32),
            pltpu.SemaphoreType.DMA,
            pltpu.SemaphoreType.DMA,
            pltpu.SemaphoreType.DMA,
            pltpu.SemaphoreType.DMA,
        ],
        compiler_params=pltpu.CompilerParams(
            use_tc_tiling_on_sc=False, needs_layout_passes=False,
            disable_bounds_checks=True),
    )
    def emb_kernel(table_hbm, idxt_hbm, pos_hbm, out_hbm,
                   idx0, idx1, rows0, rows1, tb0, tb1, pos_v,
                   gsem0, gsem1, wsem0, wsem1):
        idx_v = (idx0, idx1)
        rows_v = (rows0, rows1)
        tblk = (tb0, tb1)
        gsem = (gsem0, gsem1)
        wsem = (wsem0, wsem1)

        wid = lax.axis_index("s") * info.num_cores + lax.axis_index("c")
        b0 = wid * bpw

        pltpu.sync_copy(pos_hbm, pos_v)

        iota = lax.iota(jnp.int32, LANES)

        for b in range(2):
            pltpu.sync_copy(idxt_hbm.at[b, pl.ds(b0, bpw)], idx_v[b])
            pltpu.async_copy(table_hbm.at[idx_v[b]], rows_v[b], gsem[b])

        def pos_body(s, _):
            for b in range(2):
                c = 2 * s + b
                pltpu.make_async_copy(
                    table_hbm.at[idx_v[b]], rows_v[b], gsem[b]).wait()
                c2 = lax.min(c + 2, seq_len - 1)
                pltpu.sync_copy(idxt_hbm.at[c2, pl.ds(b0, bpw)], idx_v[b])

                @pl.when(s > 0)
                def _wait_prev_write():
                    pltpu.make_async_copy(
                        tblk[b], out_hbm.at[0, :, pl.ds(b0, bpw)],
                        wsem[b]).wait()

                pvecs = [pos_v[c, pl.ds(k * LANES, LANES)] for k in range(vecs)]
                row_ids = [iota + k * LANES for k in range(vecs)]
                zeros = iota * 0

                @plsc.parallel_loop(0, bpw, unroll=8)
                def _row(r):
                    col = zeros + r
                    for k in range(vecs):
                        val = rows_v[b][r, pl.ds(k * LANES, LANES)] + pvecs[k]
                        plsc.store_scatter(tblk[b], [row_ids[k], col], val)

                pltpu.async_copy(table_hbm.at[idx_v[b]], rows_v[b], gsem[b])
                pltpu.async_copy(
                    tblk[b], out_hbm.at[c, :, pl.ds(b0, bpw)], wsem[b])
            return _

        lax.fori_loop(0, seq_len // 2, pos_body, None)

        for b in range(2):
            pltpu.make_async_copy(
                table_hbm.at[idx_v[b]], rows_v[b], gsem[b]).wait()
            pltpu.make_async_copy(
                tblk[b], out_hbm.at[0, :, pl.ds(b0, bpw)], wsem[b]).wait()

    return emb_kernel


def kernel(inputs, token_table, pos_table):
    batch, seq_len = inputs.shape
    vocab, embed = token_table.shape
    idx_t = inputs.T.astype(jnp.int32)
    fn = _build(batch, seq_len, vocab, embed)
    out = fn(token_table, idx_t, pos_table)
    return out.transpose(2, 0, 1)


# speedup vs baseline: 1.8600x; 1.8600x over previous
import functools

import jax
import jax.numpy as jnp
from jax import lax
from jax.experimental import pallas as pl
from jax.experimental.pallas import tpu as pltpu
from jax.experimental.pallas import tpu_sc as plsc

LANES = 16


@functools.lru_cache(maxsize=None)
def _build(batch: int, seq_len: int, vocab: int, embed: int):
    info = plsc.get_sparse_core_info()
    nw = info.num_cores * info.num_subcores
    assert batch % nw == 0
    bpw = batch // nw
    vecs = embed // LANES

    mesh = plsc.VectorSubcoreMesh(core_axis_name="c", subcore_axis_name="s")

    @functools.partial(
        pl.kernel,
        out_type=jax.ShapeDtypeStruct((seq_len, embed, batch), jnp.float32),
        mesh=mesh,
        scratch_types=[
            pltpu.VMEM((bpw,), jnp.int32),
            pltpu.VMEM((bpw,), jnp.int32),
            pltpu.VMEM((bpw, embed), jnp.float32),
            pltpu.VMEM((bpw, embed), jnp.float32),
            pltpu.VMEM((embed, bpw + 1), jnp.float32),
            pltpu.VMEM((embed, bpw + 1), jnp.float32),
            pltpu.VMEM((seq_len, embed), jnp.float32),
            pltpu.SemaphoreType.DMA,
            pltpu.SemaphoreType.DMA,
            pltpu.SemaphoreType.DMA,
            pltpu.SemaphoreType.DMA,
        ],
        compiler_params=pltpu.CompilerParams(
            use_tc_tiling_on_sc=False, needs_layout_passes=False,
            disable_bounds_checks=True),
    )
    def emb_kernel(table_hbm, idxt_hbm, pos_hbm, out_hbm,
                   idx0, idx1, rows0, rows1, tb0, tb1, pos_v,
                   gsem0, gsem1, wsem0, wsem1):
        idx_v = (idx0, idx1)
        rows_v = (rows0, rows1)
        tblk = (tb0, tb1)
        gsem = (gsem0, gsem1)
        wsem = (wsem0, wsem1)

        wid = lax.axis_index("s") * info.num_cores + lax.axis_index("c")
        b0 = wid * bpw

        pltpu.sync_copy(pos_hbm, pos_v)

        iota = lax.iota(jnp.int32, LANES)

        for b in range(2):
            pltpu.sync_copy(idxt_hbm.at[b, pl.ds(b0, bpw)], idx_v[b])
            pltpu.async_copy(table_hbm.at[idx_v[b]], rows_v[b], gsem[b])

        def pos_body(s, _):
            for b in range(2):
                c = 2 * s + b
                pltpu.make_async_copy(
                    table_hbm.at[idx_v[b]], rows_v[b], gsem[b]).wait()
                c2 = lax.min(c + 2, seq_len - 1)
                pltpu.sync_copy(idxt_hbm.at[c2, pl.ds(b0, bpw)], idx_v[b])

                @pl.when(s > 0)
                def _wait_prev_write():
                    pltpu.make_async_copy(
                        tblk[b].at[:, pl.ds(0, bpw)],
                        out_hbm.at[0, :, pl.ds(b0, bpw)], wsem[b]).wait()

                pvecs = [pos_v[c, pl.ds(k * LANES, LANES)] for k in range(vecs)]
                row_ids = [iota + k * LANES for k in range(vecs)]
                zeros = iota * 0

                @plsc.parallel_loop(0, bpw, unroll=8)
                def _row(r):
                    col = zeros + r
                    for k in range(vecs):
                        val = rows_v[b][r, pl.ds(k * LANES, LANES)] + pvecs[k]
                        plsc.store_scatter(tblk[b], [row_ids[k], col], val)

                pltpu.async_copy(table_hbm.at[idx_v[b]], rows_v[b], gsem[b])
                pltpu.async_copy(
                    tblk[b].at[:, pl.ds(0, bpw)],
                    out_hbm.at[c, :, pl.ds(b0, bpw)], wsem[b])
            return _

        lax.fori_loop(0, seq_len // 2, pos_body, None)

        for b in range(2):
            pltpu.make_async_copy(
                table_hbm.at[idx_v[b]], rows_v[b], gsem[b]).wait()
            pltpu.make_async_copy(
                tblk[b].at[:, pl.ds(0, bpw)],
                out_hbm.at[0, :, pl.ds(b0, bpw)], wsem[b]).wait()

    return emb_kernel


def kernel(inputs, token_table, pos_table):
    batch, seq_len = inputs.shape
    vocab, embed = token_table.shape
    idx_t = inputs.T.astype(jnp.int32)
    fn = _build(batch, seq_len, vocab, embed)
    out = fn(token_table, idx_t, pos_table)
    return out.transpose(2, 0, 1)


# speedup vs baseline: 2.1140x; 1.1365x over previous
import functools

import jax
import jax.numpy as jnp
from jax import lax
from jax.experimental import pallas as pl
from jax.experimental.pallas import tpu as pltpu
from jax.experimental.pallas import tpu_sc as plsc

LANES = 16
SEQ_SPLIT = 2


@functools.lru_cache(maxsize=None)
def _build(batch: int, seq_len: int, vocab: int, embed: int):
    info = plsc.get_sparse_core_info()
    nw = info.num_cores * info.num_subcores
    nb = nw // SEQ_SPLIT
    assert batch % nb == 0 and seq_len % (2 * SEQ_SPLIT) == 0
    bpw = batch // nb
    spw = seq_len // SEQ_SPLIT
    vecs = embed // LANES
    pitch = bpw + 1

    mesh = plsc.VectorSubcoreMesh(core_axis_name="c", subcore_axis_name="s")

    @functools.partial(
        pl.kernel,
        out_type=jax.ShapeDtypeStruct((seq_len, embed, batch), jnp.float32),
        mesh=mesh,
        scratch_types=[
            pltpu.VMEM((bpw,), jnp.int32),
            pltpu.VMEM((bpw,), jnp.int32),
            pltpu.VMEM((bpw, embed), jnp.float32),
            pltpu.VMEM((bpw, embed), jnp.float32),
            pltpu.VMEM((embed, pitch), jnp.float32),
            pltpu.VMEM((embed, pitch), jnp.float32),
            pltpu.VMEM((seq_len, embed), jnp.float32),
            pltpu.SemaphoreType.DMA,
            pltpu.SemaphoreType.DMA,
            pltpu.SemaphoreType.DMA,
            pltpu.SemaphoreType.DMA,
        ],
        compiler_params=pltpu.CompilerParams(
            use_tc_tiling_on_sc=False, needs_layout_passes=False,
            disable_bounds_checks=True),
    )
    def emb_kernel(table_hbm, idxt_hbm, pos_hbm, out_hbm,
                   idx0, idx1, rows0, rows1, tb0, tb1, pos_v,
                   gsem0, gsem1, wsem0, wsem1):
        idx_v = (idx0, idx1)
        rows_v = (rows0, rows1)
        tblk = (tb0, tb1)
        gsem = (gsem0, gsem1)
        wsem = (wsem0, wsem1)

        wid = lax.axis_index("s") * info.num_cores + lax.axis_index("c")
        b0 = (wid % nb) * bpw
        s_lo = (wid // nb) * spw

        pltpu.sync_copy(pos_hbm, pos_v)

        iota = lax.iota(jnp.int32, LANES)

        for b in range(2):
            pltpu.sync_copy(idxt_hbm.at[s_lo + b, pl.ds(b0, bpw)], idx_v[b])
            pltpu.async_copy(table_hbm.at[idx_v[b]], rows_v[b], gsem[b])

        def pos_body(s, _):
            for b in range(2):
                c = s_lo + 2 * s + b
                pltpu.make_async_copy(
                    table_hbm.at[idx_v[b]], rows_v[b], gsem[b]).wait()
                c2 = lax.min(c + 2, s_lo + spw - 1)
                pltpu.sync_copy(idxt_hbm.at[c2, pl.ds(b0, bpw)], idx_v[b])

                @pl.when(s > 0)
                def _wait_prev_write():
                    pltpu.make_async_copy(
                        tblk[b].at[:, pl.ds(0, bpw)],
                        out_hbm.at[0, :, pl.ds(b0, bpw)], wsem[b]).wait()

                pvecs = [pos_v[c, pl.ds(k * LANES, LANES)] for k in range(vecs)]
                row_ids = [iota + k * LANES for k in range(vecs)]
                zeros = iota * 0

                @plsc.parallel_loop(0, bpw, unroll=8)
                def _row(r):
                    col = zeros + r
                    for k in range(vecs):
                        val = rows_v[b][r, pl.ds(k * LANES, LANES)] + pvecs[k]
                        plsc.store_scatter(tblk[b], [row_ids[k], col], val)

                pltpu.async_copy(table_hbm.at[idx_v[b]], rows_v[b], gsem[b])
                pltpu.async_copy(
                    tblk[b].at[:, pl.ds(0, bpw)],
                    out_hbm.at[c, :, pl.ds(b0, bpw)], wsem[b])
            return _

        lax.fori_loop(0, spw // 2, pos_body, None)

        for b in range(2):
            pltpu.make_async_copy(
                table_hbm.at[idx_v[b]], rows_v[b], gsem[b]).wait()
            pltpu.make_async_copy(
                tblk[b].at[:, pl.ds(0, bpw)],
                out_hbm.at[0, :, pl.ds(b0, bpw)], wsem[b]).wait()

    return emb_kernel


def kernel(inputs, token_table, pos_table):
    batch, seq_len = inputs.shape
    vocab, embed = token_table.shape
    idx_t = inputs.T.astype(jnp.int32)
    fn = _build(batch, seq_len, vocab, embed)
    out = fn(token_table, idx_t, pos_table)
    return out.transpose(2, 0, 1)


# speedup vs baseline: 2.6173x; 1.2381x over previous
import functools

import jax
import jax.numpy as jnp
from jax import lax
from jax.experimental import pallas as pl
from jax.experimental.pallas import tpu as pltpu
from jax.experimental.pallas import tpu_sc as plsc

LANES = 16
SEQ_SPLIT = 2
RING = 4


@functools.lru_cache(maxsize=None)
def _build(batch: int, seq_len: int, vocab: int, embed: int):
    info = plsc.get_sparse_core_info()
    nw = info.num_cores * info.num_subcores
    nb = nw // SEQ_SPLIT
    assert batch % nb == 0 and seq_len % (RING * SEQ_SPLIT) == 0
    bpw = batch // nb
    spw = seq_len // SEQ_SPLIT
    vecs = embed // LANES
    pitch = bpw + 1

    mesh = plsc.VectorSubcoreMesh(core_axis_name="c", subcore_axis_name="s")

    @functools.partial(
        pl.kernel,
        out_type=jax.ShapeDtypeStruct((seq_len, embed, batch), jnp.float32),
        mesh=mesh,
        scratch_types=(
            [pltpu.VMEM((bpw,), jnp.int32) for _ in range(RING)]
            + [pltpu.VMEM((bpw, embed), jnp.float32) for _ in range(RING)]
            + [pltpu.VMEM((embed, pitch), jnp.float32) for _ in range(2)]
            + [pltpu.VMEM((seq_len, embed), jnp.float32)]
            + [pltpu.SemaphoreType.DMA for _ in range(2 * RING + 2)]
        ),
        compiler_params=pltpu.CompilerParams(
            use_tc_tiling_on_sc=False, needs_layout_passes=False,
            disable_bounds_checks=True),
    )
    def emb_kernel(table_hbm, idxt_hbm, pos_hbm, out_hbm, *scratch):
        idx_v = scratch[:RING]
        rows_v = scratch[RING:2 * RING]
        tblk = scratch[2 * RING:2 * RING + 2]
        pos_v = scratch[2 * RING + 2]
        gsem = scratch[2 * RING + 3:3 * RING + 3]
        isem = scratch[3 * RING + 3:4 * RING + 3]
        wsem = scratch[4 * RING + 3:]

        wid = lax.axis_index("s") * info.num_cores + lax.axis_index("c")
        b0 = (wid % nb) * bpw
        s_lo = (wid // nb) * spw

        pltpu.sync_copy(pos_hbm, pos_v)

        iota = lax.iota(jnp.int32, LANES)

        for b in range(RING):
            pltpu.sync_copy(idxt_hbm.at[s_lo + b, pl.ds(b0, bpw)], idx_v[b])
            pltpu.async_copy(table_hbm.at[idx_v[b]], rows_v[b], gsem[b])

        def pos_body(i, _):
            for b in range(RING):
                c = s_lo + RING * i + b
                w = b % 2
                pltpu.make_async_copy(
                    table_hbm.at[idx_v[b]], rows_v[b], gsem[b]).wait()
                c2 = lax.min(c + RING, s_lo + spw - 1)
                pltpu.async_copy(
                    idxt_hbm.at[c2, pl.ds(b0, bpw)], idx_v[b], isem[b])

                def _wait_prev_write():
                    pltpu.make_async_copy(
                        tblk[w].at[:, pl.ds(0, bpw)],
                        out_hbm.at[0, :, pl.ds(b0, bpw)], wsem[w]).wait()

                if b < 2:
                    pl.when(i > 0)(_wait_prev_write)
                else:
                    _wait_prev_write()

                pvecs = [pos_v[c, pl.ds(k * LANES, LANES)] for k in range(vecs)]
                row_ids = [iota + k * LANES for k in range(vecs)]

                @plsc.parallel_loop(0, bpw, unroll=8, carry=iota * 0)
                def _row(r, col):
                    for k in range(vecs):
                        val = rows_v[b][r, pl.ds(k * LANES, LANES)] + pvecs[k]
                        plsc.store_scatter(tblk[w], [row_ids[k], col], val)
                    return col + 1

                pltpu.make_async_copy(
                    idxt_hbm.at[0, pl.ds(b0, bpw)], idx_v[b], isem[b]).wait()
                pltpu.async_copy(table_hbm.at[idx_v[b]], rows_v[b], gsem[b])
                pltpu.async_copy(
                    tblk[w].at[:, pl.ds(0, bpw)],
                    out_hbm.at[c, :, pl.ds(b0, bpw)], wsem[w])
            return _

        lax.fori_loop(0, spw // RING, pos_body, None)

        for b in range(RING):
            pltpu.make_async_copy(
                table_hbm.at[idx_v[b]], rows_v[b], gsem[b]).wait()
        for w in range(2):
            pltpu.make_async_copy(
                tblk[w].at[:, pl.ds(0, bpw)],
                out_hbm.at[0, :, pl.ds(b0, bpw)], wsem[w]).wait()

    return emb_kernel


def kernel(inputs, token_table, pos_table):
    batch, seq_len = inputs.shape
    vocab, embed = token_table.shape
    idx_t = inputs.T.astype(jnp.int32)
    fn = _build(batch, seq_len, vocab, embed)
    out = fn(token_table, idx_t, pos_table)
    return out.transpose(2, 0, 1)


# speedup vs baseline: 2.6284x; 1.0042x over previous
import functools

import jax
import jax.numpy as jnp
from jax import lax
from jax.experimental import pallas as pl
from jax.experimental.pallas import tpu as pltpu
from jax.experimental.pallas import tpu_sc as plsc

LANES = 16
SEQ_SPLIT = 2
RING = 4


@functools.lru_cache(maxsize=None)
def _build(batch: int, seq_len: int, vocab: int, embed: int):
    info = plsc.get_sparse_core_info()
    nw = info.num_cores * info.num_subcores
    nb = nw // SEQ_SPLIT
    assert batch % nb == 0 and seq_len % (RING * SEQ_SPLIT) == 0
    bpw = batch // nb
    spw = seq_len // SEQ_SPLIT
    vecs = embed // LANES
    pitch = bpw + 1

    mesh = plsc.VectorSubcoreMesh(core_axis_name="c", subcore_axis_name="s")

    @functools.partial(
        pl.kernel,
        out_type=jax.ShapeDtypeStruct((seq_len, embed, batch), jnp.float32),
        mesh=mesh,
        scratch_types=(
            [pltpu.VMEM((bpw,), jnp.int32) for _ in range(RING)]
            + [pltpu.VMEM((bpw, embed), jnp.float32) for _ in range(RING)]
            + [pltpu.VMEM((embed, pitch), jnp.float32) for _ in range(2)]
            + [pltpu.VMEM((seq_len, embed), jnp.float32)]
            + [pltpu.SemaphoreType.DMA for _ in range(2 * RING + 2)]
        ),
        compiler_params=pltpu.CompilerParams(
            use_tc_tiling_on_sc=False, needs_layout_passes=False,
            disable_bounds_checks=True),
    )
    def emb_kernel(table_hbm, idxt_hbm, pos_hbm, out_hbm, *scratch):
        idx_v = scratch[:RING]
        rows_v = scratch[RING:2 * RING]
        tblk = scratch[2 * RING:2 * RING + 2]
        pos_v = scratch[2 * RING + 2]
        gsem = scratch[2 * RING + 3:3 * RING + 3]
        isem = scratch[3 * RING + 3:4 * RING + 3]
        wsem = scratch[4 * RING + 3:]

        wid = lax.axis_index("s") * info.num_cores + lax.axis_index("c")
        b0 = (wid % nb) * bpw
        s_lo = (wid // nb) * spw

        pltpu.sync_copy(pos_hbm, pos_v)

        iota = lax.iota(jnp.int32, LANES)

        for b in range(RING):
            pltpu.sync_copy(idxt_hbm.at[s_lo + b, pl.ds(b0, bpw)], idx_v[b])
            pltpu.async_copy(table_hbm.at[idx_v[b]], rows_v[b], gsem[b])

        def pos_body(i, _):
            for b in range(RING):
                c = s_lo + RING * i + b
                w = b % 2
                pltpu.make_async_copy(
                    table_hbm.at[idx_v[b]], rows_v[b], gsem[b]).wait()
                c2 = lax.min(c + RING, s_lo + spw - 1)
                pltpu.async_copy(
                    idxt_hbm.at[c2, pl.ds(b0, bpw)], idx_v[b], isem[b])

                def _wait_prev_write():
                    pltpu.make_async_copy(
                        tblk[w].at[:, pl.ds(0, bpw)],
                        out_hbm.at[0, :, pl.ds(b0, bpw)], wsem[w]).wait()

                if b < 2:
                    pl.when(i > 0)(_wait_prev_write)
                else:
                    _wait_prev_write()

                pvecs = [pos_v[c, pl.ds(k * LANES, LANES)] for k in range(vecs)]
                row_ids = [iota + k * LANES for k in range(vecs)]

                @plsc.parallel_loop(0, bpw, unroll=16, carry=iota * 0)
                def _row(r, col):
                    for k in range(vecs):
                        val = rows_v[b][r, pl.ds(k * LANES, LANES)] + pvecs[k]
                        plsc.store_scatter(tblk[w], [row_ids[k], col], val)
                    return col + 1

                pltpu.make_async_copy(
                    idxt_hbm.at[0, pl.ds(b0, bpw)], idx_v[b], isem[b]).wait()
                pltpu.async_copy(table_hbm.at[idx_v[b]], rows_v[b], gsem[b])
                pltpu.async_copy(
                    tblk[w].at[:, pl.ds(0, bpw)],
                    out_hbm.at[c, :, pl.ds(b0, bpw)], wsem[w])
            return _

        lax.fori_loop(0, spw // RING, pos_body, None)

        for b in range(RING):
            pltpu.make_async_copy(
                table_hbm.at[idx_v[b]], rows_v[b], gsem[b]).wait()
        for w in range(2):
            pltpu.make_async_copy(
                tblk[w].at[:, pl.ds(0, bpw)],
                out_hbm.at[0, :, pl.ds(b0, bpw)], wsem[w]).wait()

    return emb_kernel


def kernel(inputs, token_table, pos_table):
    batch, seq_len = inputs.shape
    vocab, embed = token_table.shape
    idx_t = inputs.T.astype(jnp.int32)
    fn = _build(batch, seq_len, vocab, embed)
    out = fn(token_table, idx_t, pos_table)
    return out.transpose(2, 0, 1)
